# Initial kernel scaffold; baseline (speedup 1.0000x reference)
import jax
import jax.numpy as jnp
from jax import lax
from jax.experimental import pallas as pl
from jax.experimental.pallas import tpu as pltpu
from jax.experimental.pallas import tpu_sc as plsc

N0 = 4096
NUM_OMICS = 3
NN = N0 * NUM_OMICS
EE = 393216
RR = 6
DD = 128
SQK_STRIDE = 16

NUM_SC = 2
NUM_TILES = 16
NW = NUM_SC * NUM_TILES
TILE_E = EE // NW

E1_BLK = 1024
E2_BLK = 128
NPT = NN // NUM_TILES


def _elu(x):
    return jnp.where(x > 0, x, jnp.exp(jnp.minimum(x, 0.0)) - 1.0)


def _proj_body(x_ref, pw_ref, pb_ref, s1w_ref, s1b_ref, xh_ref, sl_ref):
    p = jnp.dot(x_ref[...], pw_ref[...], preferred_element_type=jnp.float32)
    p = p + pb_ref[...]
    xh = _elu(p)
    xh_ref[...] = xh
    sl_ref[...] = (
        jnp.dot(xh, s1w_ref[...], preferred_element_type=jnp.float32) + s1b_ref[...]
    )


def _proj(x, pw, pb, s1w, s1b):
    n, k = x.shape
    bn = 256
    return pl.pallas_call(
        _proj_body,
        grid=(n // bn,),
        in_specs=[
            pl.BlockSpec((bn, k), lambda b: (b, 0)),
            pl.BlockSpec((k, DD), lambda b: (0, 0)),
            pl.BlockSpec((1, DD), lambda b: (0, 0)),
            pl.BlockSpec((DD, DD), lambda b: (0, 0)),
            pl.BlockSpec((1, DD), lambda b: (0, 0)),
        ],
        out_specs=[
            pl.BlockSpec((bn, DD), lambda b: (b, 0)),
            pl.BlockSpec((bn, DD), lambda b: (b, 0)),
        ],
        out_shape=[
            jax.ShapeDtypeStruct((n, DD), jnp.float32),
            jax.ShapeDtypeStruct((n, DD), jnp.float32),
        ],
    )(x, pw, pb.reshape(1, DD), s1w, s1b.reshape(1, DD))


def _hall_body(x_ref, w_ref, vqk_ref, h_ref, sqk_ref):
    r = pl.program_id(0)
    h = jnp.dot(x_ref[...], w_ref[0], preferred_element_type=jnp.float32)
    h_ref[0] = h

    @pl.when(r == 0)
    def _():
        sqk_ref[...] = jnp.dot(
            x_ref[...], vqk_ref[...], preferred_element_type=jnp.float32
        )


def _hall_scores(x, w, vqk):
    bn = 256
    return pl.pallas_call(
        _hall_body,
        grid=(RR, NN // bn),
        in_specs=[
            pl.BlockSpec((bn, DD), lambda r, b: (b, 0)),
            pl.BlockSpec((1, DD, DD), lambda r, b: (r, 0, 0)),
            pl.BlockSpec((DD, SQK_STRIDE), lambda r, b: (0, 0)),
        ],
        out_specs=[
            pl.BlockSpec((1, bn, DD), lambda r, b: (r, b, 0)),
            pl.BlockSpec((bn, SQK_STRIDE), lambda r, b: (b, 0)),
        ],
        out_shape=[
            jax.ShapeDtypeStruct((RR, NN, DD), jnp.float32),
            jax.ShapeDtypeStruct((NN, SQK_STRIDE), jnp.float32),
        ],
    )(x, w, vqk)


def _combine_body(op_ref, sp_ref, dp_ref, b_ref, sl_ref, out_ref):
    s = sp_ref[0] + sp_ref[1]
    deg = dp_ref[0] + dp_ref[1]
    w = deg / (s + 1e-16)
    h = w * (op_ref[0] + op_ref[1]) + b_ref[...] + sl_ref[...]
    out_ref[...] = _elu(h)


def _combine(outp, s_part, deg_part, b, sl):
    bn = 256
    return pl.pallas_call(
        _combine_body,
        grid=(NN // bn,),
        in_specs=[
            pl.BlockSpec((2, bn, DD), lambda b_: (0, b_, 0)),
            pl.BlockSpec((2, bn, 1), lambda b_: (0, b_, 0)),
            pl.BlockSpec((2, bn, 1), lambda b_: (0, b_, 0)),
            pl.BlockSpec((1, DD), lambda b_: (0, 0)),
            pl.BlockSpec((bn, DD), lambda b_: (b_, 0)),
        ],
        out_specs=pl.BlockSpec((bn, DD), lambda b_: (b_, 0)),
        out_shape=jax.ShapeDtypeStruct((NN, DD), jnp.float32),
    )(
        outp,
        s_part.reshape(2, NN, 1),
        deg_part.reshape(2, NN, 1),
        b.reshape(1, DD),
        sl,
    )


def _head_body(
    op_ref, sp_ref, dp_ref, b_ref, sl_ref, l1w_ref, l1b_ref, l2w_ref, l2b_ref, y_ref
):
    s = sp_ref[0] + sp_ref[1]
    deg = dp_ref[0] + dp_ref[1]
    w = deg / (s + 1e-16)
    h = _elu(w * (op_ref[0] + op_ref[1]) + b_ref[...] + sl_ref[...])
    z = _elu(jnp.dot(h, l1w_ref[...], preferred_element_type=jnp.float32) + l1b_ref[...])
    y_ref[...] = (
        jnp.dot(z, l2w_ref[...], preferred_element_type=jnp.float32) + l2b_ref[...]
    )


def _head(outp, s_part, deg_part, b, sl, l1w, l1b, l2w_pad, l2b_pad):
    bn = 256
    return pl.pallas_call(
        _head_body,
        grid=(N0 // bn,),
        in_specs=[
            pl.BlockSpec((2, bn, DD), lambda b_: (0, b_, 0)),
            pl.BlockSpec((2, bn, 1), lambda b_: (0, b_, 0)),
            pl.BlockSpec((2, bn, 1), lambda b_: (0, b_, 0)),
            pl.BlockSpec((1, DD), lambda b_: (0, 0)),
            pl.BlockSpec((bn, DD), lambda b_: (b_, 0)),
            pl.BlockSpec((DD, DD), lambda b_: (0, 0)),
            pl.BlockSpec((1, DD), lambda b_: (0, 0)),
            pl.BlockSpec((DD, DD), lambda b_: (0, 0)),
            pl.BlockSpec((1, DD), lambda b_: (0, 0)),
        ],
        out_specs=pl.BlockSpec((bn, DD), lambda b_: (b_, 0)),
        out_shape=jax.ShapeDtypeStruct((N0, DD), jnp.float32),
    )(
        outp,
        s_part.reshape(2, NN, 1),
        deg_part.reshape(2, NN, 1),
        b.reshape(1, DD),
        sl,
        l1w,
        l1b.reshape(1, DD),
        l2w_pad,
        l2b_pad,
    )


_SC_MESH = plsc.VectorSubcoreMesh(core_axis_name="c", subcore_axis_name="s")


def _zero_fill(buf, nwords):
    z = jnp.zeros((16,), jnp.float32)

    def body(i, _):
        buf[pl.ds(i * 16, 16)] = z
        return 0

    lax.fori_loop(0, nwords // 16, body, 0)


def _e1_body(
    sqk_hbm, src_hbm, dst_hbm, et_hbm,
    e_hbm, s_out, deg_out,
    src_v, dst_v, et_v, idxd_v, idxs_v, dstw_v,
    val_d, val_s, e_v, ones_v, zbuf,
    s_sh, deg_sh, sem,
):
    cid = lax.axis_index("c")
    sid = lax.axis_index("s")
    base0 = (cid * NUM_TILES + sid) * TILE_E

    _zero_fill(zbuf, NPT)
    pltpu.sync_copy(zbuf, s_sh.at[pl.ds(sid * NPT, NPT)])
    pltpu.sync_copy(zbuf, deg_sh.at[pl.ds(sid * NPT, NPT)])

    def fill_ones(i, _):
        ones_v[pl.ds(i * 16, 16)] = jnp.ones((16,), jnp.float32)
        return 0

    lax.fori_loop(0, E1_BLK // 16, fill_ones, 0)
    plsc.subcore_barrier()

    def blk_body(blk, _):
        base = base0 + blk * E1_BLK
        pltpu.sync_copy(src_hbm.at[pl.ds(base, E1_BLK)], src_v)
        pltpu.sync_copy(dst_hbm.at[pl.ds(base, E1_BLK)], dst_v)
        pltpu.sync_copy(et_hbm.at[pl.ds(base, E1_BLK)], et_v)

        def idx_body(i, _):
            sl = pl.ds(i * 16, 16)
            e16 = et_v[sl]
            d16 = dst_v[sl]
            s16 = src_v[sl]
            r = i // 8
            o = (i % 8) * 16
            idxd_v[r, pl.ds(o, 16)] = d16 * SQK_STRIDE + e16
            idxs_v[r, pl.ds(o, 16)] = s16 * SQK_STRIDE + 8 + e16
            dstw_v[r, pl.ds(o, 16)] = d16
            return 0

        lax.fori_loop(0, E1_BLK // 16, idx_body, 0)

        copies = []
        for c in range(E1_BLK // 128):
            copies.append(
                pltpu.async_copy(
                    sqk_hbm.at[idxd_v.at[c]], val_d.at[pl.ds(c * 128, 128)], sem
                )
            )
            copies.append(
                pltpu.async_copy(
                    sqk_hbm.at[idxs_v.at[c]], val_s.at[pl.ds(c * 128, 128)], sem
                )
            )
        for cp in copies:
            cp.wait()

        def logit_body(i, _):
            sl = pl.ds(i * 16, 16)
            l = val_d[sl] + val_s[sl]
            l = jnp.maximum(l, 0.2 * l)
            e_v[sl] = jnp.exp(l)
            return 0

        lax.fori_loop(0, E1_BLK // 16, logit_body, 0)

        pltpu.sync_copy(e_v, e_hbm.at[pl.ds(base, E1_BLK)])

        copies = []
        for c in range(E1_BLK // 128):
            sl = pl.ds(c * 128, 128)
            copies.append(
                pltpu.async_copy(e_v.at[sl], s_sh.at[dstw_v.at[c]], sem, add=True)
            )
            copies.append(
                pltpu.async_copy(ones_v.at[sl], deg_sh.at[dstw_v.at[c]], sem, add=True)
            )
        for cp in copies:
            cp.wait()
        return 0

    lax.fori_loop(0, TILE_E // E1_BLK, blk_body, 0)
    plsc.subcore_barrier()

    sl = pl.ds(sid * NPT, NPT)
    pltpu.sync_copy(s_sh.at[sl], s_out.at[cid, sl])
    pltpu.sync_copy(deg_sh.at[sl], deg_out.at[cid, sl])


_e1_call = pl.kernel(
    _e1_body,
    out_type=[
        jax.ShapeDtypeStruct((EE,), jnp.float32),
        jax.ShapeDtypeStruct((NUM_SC, NN), jnp.float32),
        jax.ShapeDtypeStruct((NUM_SC, NN), jnp.float32),
    ],
    mesh=_SC_MESH,
    scratch_types=[
        pltpu.VMEM((E1_BLK,), jnp.int32),
        pltpu.VMEM((E1_BLK,), jnp.int32),
        pltpu.VMEM((E1_BLK,), jnp.int32),
        pltpu.VMEM((E1_BLK // 128, 128), jnp.int32),
        pltpu.VMEM((E1_BLK // 128, 128), jnp.int32),
        pltpu.VMEM((E1_BLK // 128, 128), jnp.int32),
        pltpu.VMEM((E1_BLK,), jnp.float32),
        pltpu.VMEM((E1_BLK,), jnp.float32),
        pltpu.VMEM((E1_BLK,), jnp.float32),
        pltpu.VMEM((E1_BLK,), jnp.float32),
        pltpu.VMEM((NPT,), jnp.float32),
        pltpu.VMEM_SHARED((NN,), jnp.float32),
        pltpu.VMEM_SHARED((NN,), jnp.float32),
        pltpu.SemaphoreType.DMA,
    ],
)


def _e2_body(
    hall_hbm, src_hbm, dst_hbm, et_hbm, e_hbm,
    out_hbm,
    src_v, dst_v, et_v, idxh_v, dstw_v, e_v, rows_v, zbuf,
    out_sh, sem,
):
    cid = lax.axis_index("c")
    sid = lax.axis_index("s")
    base0 = (cid * NUM_TILES + sid) * TILE_E

    _zero_fill(zbuf, 2048)
    nz = NPT * DD // 2048

    def zc_body(i, _):
        pltpu.sync_copy(zbuf, out_sh.at[pl.ds(sid * NPT * DD + i * 2048, 2048)])
        return 0

    lax.fori_loop(0, nz, zc_body, 0)
    plsc.subcore_barrier()

    def blk_body(blk, _):
        base = base0 + blk * E2_BLK
        pltpu.sync_copy(src_hbm.at[pl.ds(base, E2_BLK)], src_v)
        pltpu.sync_copy(dst_hbm.at[pl.ds(base, E2_BLK)], dst_v)
        pltpu.sync_copy(et_hbm.at[pl.ds(base, E2_BLK)], et_v)
        pltpu.sync_copy(e_hbm.at[pl.ds(base, E2_BLK)], e_v)

        def idx_body(i, _):
            sl = pl.ds(i * 16, 16)
            idxh_v[0, sl] = et_v[sl] * NN + src_v[sl]
            dstw_v[0, sl] = dst_v[sl]
            return 0

        lax.fori_loop(0, E2_BLK // 16, idx_body, 0)

        pltpu.async_copy(hall_hbm.at[idxh_v.at[0]], rows_v, sem).wait()

        def scale_body(b, _):
            cf = plsc.load_gather(e_v, [jnp.full((16,), b, jnp.int32)])
            for j in range(DD // 16):
                sl2 = pl.ds(b * DD + j * 16, 16)
                rows_v[sl2] = rows_v[sl2] * cf
            return 0

        lax.fori_loop(0, E2_BLK, scale_body, 0)

        pltpu.sync_copy(rows2_view(rows_v), out_sh.at[dstw_v.at[0]], add=True)
        return 0

    lax.fori_loop(0, TILE_E // E2_BLK, blk_body, 0)
    plsc.subcore_barrier()

    pltpu.sync_copy(
        out_sh.at[pl.ds(sid * NPT * DD, NPT * DD)],
        out_hbm.at[cid, pl.ds(sid * NPT * DD, NPT * DD)],
    )


def rows2_view(rows_v):
    return rows_v.reshape(E2_BLK, DD)


_e2_call = pl.kernel(
    _e2_body,
    out_type=[
        jax.ShapeDtypeStruct((NUM_SC, NN * DD), jnp.float32),
    ],
    mesh=_SC_MESH,
    scratch_types=[
        pltpu.VMEM((E2_BLK,), jnp.int32),
        pltpu.VMEM((E2_BLK,), jnp.int32),
        pltpu.VMEM((E2_BLK,), jnp.int32),
        pltpu.VMEM((1, E2_BLK), jnp.int32),
        pltpu.VMEM((1, E2_BLK), jnp.int32),
        pltpu.VMEM((E2_BLK,), jnp.float32),
        pltpu.VMEM((E2_BLK * DD,), jnp.float32),
        pltpu.VMEM((2048,), jnp.float32),
        pltpu.VMEM_SHARED((NN * DD,), jnp.float32),
        pltpu.SemaphoreType.DMA,
    ],
)


def kernel(
    x0, x1, x2, Pw0, Pb0, Pw1, Pb1, Pw2, Pb2,
    W1, q1, k1, b1, W2, q2, k2, b2,
    S1w, S1b, L1w, L1b, L2w, L2b, edge_index, edge_type,
):
    src = edge_index[0]
    dst = edge_index[1]
    et = edge_type

    def vqk(w, q, k):
        vq = jnp.einsum("rdo,ro->dr", w, q)
        vk = jnp.einsum("rdo,ro->dr", w, k)
        pad = jnp.zeros((DD, 2), jnp.float32)
        return jnp.concatenate([vq, pad, vk, pad], axis=1)

    vqk1 = vqk(W1, q1, k1)
    vqk2 = vqk(W2, q2, k2)
    l2w_pad = jnp.pad(L2w, ((0, 0), (0, DD - L2w.shape[1])))
    l2b_pad = jnp.pad(L2b, (0, DD - L2b.shape[0]))

    xh0, sl0 = _proj(x0, Pw0, Pb0, S1w[0], S1b[0])
    xh1, sl1 = _proj(x1, Pw1, Pb1, S1w[1], S1b[1])
    xh2, sl2 = _proj(x2, Pw2, Pb2, S1w[2], S1b[2])
    xh = jnp.concatenate([xh0, xh1, xh2], axis=0)
    sl = jnp.concatenate([sl0, sl1, sl2], axis=0)

    hall1, sqk1 = _hall_scores(xh, W1, vqk1)
    e1, s1p, degp = _e1_call(sqk1.reshape(NN * SQK_STRIDE), src, dst, et)
    (op1,) = _e2_call(hall1.reshape(RR * NN, DD), src, dst, et, e1)
    h1 = _combine(op1.reshape(NUM_SC, NN, DD), s1p, degp, b1, sl)

    hall2, sqk2 = _hall_scores(h1, W2, vqk2)
    e2, s2p, _deg2 = _e1_call(sqk2.reshape(NN * SQK_STRIDE), src, dst, et)
    (op2,) = _e2_call(hall2.reshape(RR * NN, DD), src, dst, et, e2)

    y = _head(
        op2.reshape(NUM_SC, NN, DD), s2p, degp, b2, sl, L1w, L1b, l2w_pad, l2b_pad
    )
    return y[:, : L2w.shape[1]]


# speedup vs baseline: 33.5053x; 33.5053x over previous
import jax
import jax.numpy as jnp
from jax import lax
from jax.experimental import pallas as pl
from jax.experimental.pallas import tpu as pltpu
from jax.experimental.pallas import tpu_sc as plsc

N0 = 4096
NUM_OMICS = 3
NN = N0 * NUM_OMICS
EE = 393216
RR = 6
DD = 128
SQK_STRIDE = 16

NUM_SC = 2
NUM_TILES = 16
NW = NUM_SC * NUM_TILES
TILE_E = EE // NW

E1_BLK = 1024
E2_BLK = 128
NPT = NN // NUM_TILES


def _elu(x):
    return jnp.where(x > 0, x, jnp.exp(jnp.minimum(x, 0.0)) - 1.0)


def _proj_body(x_ref, pw_ref, pb_ref, s1w_ref, s1b_ref, xh_ref, sl_ref):
    p = jnp.dot(x_ref[...], pw_ref[...], preferred_element_type=jnp.float32)
    p = p + pb_ref[...]
    xh = _elu(p)
    xh_ref[...] = xh
    sl_ref[...] = (
        jnp.dot(xh, s1w_ref[...], preferred_element_type=jnp.float32) + s1b_ref[...]
    )


def _proj(x, pw, pb, s1w, s1b):
    n, k = x.shape
    bn = 256
    return pl.pallas_call(
        _proj_body,
        grid=(n // bn,),
        in_specs=[
            pl.BlockSpec((bn, k), lambda b: (b, 0)),
            pl.BlockSpec((k, DD), lambda b: (0, 0)),
            pl.BlockSpec((1, DD), lambda b: (0, 0)),
            pl.BlockSpec((DD, DD), lambda b: (0, 0)),
            pl.BlockSpec((1, DD), lambda b: (0, 0)),
        ],
        out_specs=[
            pl.BlockSpec((bn, DD), lambda b: (b, 0)),
            pl.BlockSpec((bn, DD), lambda b: (b, 0)),
        ],
        out_shape=[
            jax.ShapeDtypeStruct((n, DD), jnp.float32),
            jax.ShapeDtypeStruct((n, DD), jnp.float32),
        ],
    )(x, pw, pb.reshape(1, DD), s1w, s1b.reshape(1, DD))


def _hall_body(x_ref, w_ref, q_ref, k_ref, h_ref, sq_ref, sk_ref):
    h = jnp.dot(x_ref[...], w_ref[0], preferred_element_type=jnp.float32)
    h_ref[0] = h
    sq_ref[0] = jnp.sum(h * q_ref[0], axis=1, keepdims=True)
    sk_ref[0] = jnp.sum(h * k_ref[0], axis=1, keepdims=True)


def _hall_scores(x, w, q, k):
    bn = 256
    return pl.pallas_call(
        _hall_body,
        grid=(RR, NN // bn),
        in_specs=[
            pl.BlockSpec((bn, DD), lambda r, b: (b, 0)),
            pl.BlockSpec((1, DD, DD), lambda r, b: (r, 0, 0)),
            pl.BlockSpec((1, 1, DD), lambda r, b: (r, 0, 0)),
            pl.BlockSpec((1, 1, DD), lambda r, b: (r, 0, 0)),
        ],
        out_specs=[
            pl.BlockSpec((1, bn, DD), lambda r, b: (r, b, 0)),
            pl.BlockSpec((1, bn, 1), lambda r, b: (r, b, 0)),
            pl.BlockSpec((1, bn, 1), lambda r, b: (r, b, 0)),
        ],
        out_shape=[
            jax.ShapeDtypeStruct((RR, NN, DD), jnp.float32),
            jax.ShapeDtypeStruct((RR, NN, 1), jnp.float32),
            jax.ShapeDtypeStruct((RR, NN, 1), jnp.float32),
        ],
    )(x, w, q.reshape(RR, 1, DD), k.reshape(RR, 1, DD))


def _combine_body(op_ref, sp_ref, dp_ref, b_ref, sl_ref, out_ref):
    s = sp_ref[0] + sp_ref[1]
    deg = dp_ref[0] + dp_ref[1]
    w = deg / (s + 1e-16)
    h = w * (op_ref[0] + op_ref[1]) + b_ref[...] + sl_ref[...]
    out_ref[...] = _elu(h)


def _combine(outp, s_part, deg_part, b, sl):
    bn = 256
    return pl.pallas_call(
        _combine_body,
        grid=(NN // bn,),
        in_specs=[
            pl.BlockSpec((2, bn, DD), lambda b_: (0, b_, 0)),
            pl.BlockSpec((2, bn, 1), lambda b_: (0, b_, 0)),
            pl.BlockSpec((2, bn, 1), lambda b_: (0, b_, 0)),
            pl.BlockSpec((1, DD), lambda b_: (0, 0)),
            pl.BlockSpec((bn, DD), lambda b_: (b_, 0)),
        ],
        out_specs=pl.BlockSpec((bn, DD), lambda b_: (b_, 0)),
        out_shape=jax.ShapeDtypeStruct((NN, DD), jnp.float32),
    )(
        outp,
        s_part.reshape(2, NN, 1),
        deg_part.reshape(2, NN, 1),
        b.reshape(1, DD),
        sl,
    )


def _head_body(
    op_ref, sp_ref, dp_ref, b_ref, sl_ref, l1w_ref, l1b_ref, l2w_ref, l2b_ref, y_ref
):
    s = sp_ref[0] + sp_ref[1]
    deg = dp_ref[0] + dp_ref[1]
    w = deg / (s + 1e-16)
    h = _elu(w * (op_ref[0] + op_ref[1]) + b_ref[...] + sl_ref[...])
    z = _elu(jnp.dot(h, l1w_ref[...], preferred_element_type=jnp.float32) + l1b_ref[...])
    y_ref[...] = (
        jnp.dot(z, l2w_ref[...], preferred_element_type=jnp.float32) + l2b_ref[...]
    )


def _head(outp, s_part, deg_part, b, sl, l1w, l1b, l2w_pad, l2b_pad):
    bn = 256
    return pl.pallas_call(
        _head_body,
        grid=(N0 // bn,),
        in_specs=[
            pl.BlockSpec((2, bn, DD), lambda b_: (0, b_, 0)),
            pl.BlockSpec((2, bn, 1), lambda b_: (0, b_, 0)),
            pl.BlockSpec((2, bn, 1), lambda b_: (0, b_, 0)),
            pl.BlockSpec((1, DD), lambda b_: (0, 0)),
            pl.BlockSpec((bn, DD), lambda b_: (b_, 0)),
            pl.BlockSpec((DD, DD), lambda b_: (0, 0)),
            pl.BlockSpec((1, DD), lambda b_: (0, 0)),
            pl.BlockSpec((DD, DD), lambda b_: (0, 0)),
            pl.BlockSpec((1, DD), lambda b_: (0, 0)),
        ],
        out_specs=pl.BlockSpec((bn, DD), lambda b_: (b_, 0)),
        out_shape=jax.ShapeDtypeStruct((N0, DD), jnp.float32),
    )(
        outp,
        s_part.reshape(2, NN, 1),
        deg_part.reshape(2, NN, 1),
        b.reshape(1, DD),
        sl,
        l1w,
        l1b.reshape(1, DD),
        l2w_pad,
        l2b_pad.reshape(1, DD),
    )


def _zero_fill(buf, nwords):
    z = jnp.zeros((16,), jnp.float32)

    def body(i, _):
        buf[pl.ds(i * 16, 16)] = z
        return 0

    lax.fori_loop(0, nwords // 16, body, 0)


def _e1_body(
    sq_hbm, sk_hbm, src_hbm, dst_hbm, et_hbm,
    e_hbm, s_out, deg_out,
    src_v, dst_v, et_v, idxd_v, idxs_v, dstw_v,
    val_d, val_s, e_v, ones_v, zbuf,
    s_sh, deg_sh, sem,
):
    cid = lax.axis_index("c")
    sid = lax.axis_index("s")
    base0 = (cid * NUM_TILES + sid) * TILE_E

    _zero_fill(zbuf, NPT)
    pltpu.sync_copy(zbuf, s_sh.at[pl.ds(sid * NPT, NPT)])
    pltpu.sync_copy(zbuf, deg_sh.at[pl.ds(sid * NPT, NPT)])

    def fill_ones(i, _):
        ones_v[pl.ds(i * 16, 16)] = jnp.ones((16,), jnp.float32)
        return 0

    lax.fori_loop(0, E1_BLK // 16, fill_ones, 0)
    plsc.subcore_barrier()

    def blk_body(blk, _):
        base = base0 + blk * E1_BLK
        pltpu.sync_copy(src_hbm.at[pl.ds(base, E1_BLK)], src_v)
        pltpu.sync_copy(dst_hbm.at[pl.ds(base, E1_BLK)], dst_v)
        pltpu.sync_copy(et_hbm.at[pl.ds(base, E1_BLK)], et_v)

        def idx_body(i, _):
            sl = pl.ds(i * 16, 16)
            e16 = et_v[sl]
            d16 = dst_v[sl]
            s16 = src_v[sl]
            r = i // 8
            o = (i % 8) * 16
            enn = e16 * NN
            idxd_v[r, pl.ds(o, 16)] = enn + d16
            idxs_v[r, pl.ds(o, 16)] = enn + s16
            dstw_v[r, pl.ds(o, 16)] = d16
            return 0

        lax.fori_loop(0, E1_BLK // 16, idx_body, 0)

        copies = []
        for c in range(E1_BLK // 128):
            copies.append(
                pltpu.async_copy(
                    sq_hbm.at[idxd_v.at[c]], val_d.at[pl.ds(c * 128, 128)], sem
                )
            )
            copies.append(
                pltpu.async_copy(
                    sk_hbm.at[idxs_v.at[c]], val_s.at[pl.ds(c * 128, 128)], sem
                )
            )
        for cp in copies:
            cp.wait()

        def logit_body(i, _):
            sl = pl.ds(i * 16, 16)
            l = val_d[sl] + val_s[sl]
            l = jnp.maximum(l, 0.2 * l)
            e_v[sl] = jnp.exp(l)
            return 0

        lax.fori_loop(0, E1_BLK // 16, logit_body, 0)

        pltpu.sync_copy(e_v, e_hbm.at[pl.ds(base, E1_BLK)])

        copies = []
        for c in range(E1_BLK // 128):
            sl = pl.ds(c * 128, 128)
            copies.append(
                pltpu.async_copy(e_v.at[sl], s_sh.at[dstw_v.at[c]], sem, add=True)
            )
            copies.append(
                pltpu.async_copy(ones_v.at[sl], deg_sh.at[dstw_v.at[c]], sem, add=True)
            )
        for cp in copies:
            cp.wait()
        return 0

    lax.fori_loop(0, TILE_E // E1_BLK, blk_body, 0)
    plsc.subcore_barrier()

    sl = pl.ds(sid * NPT, NPT)
    pltpu.sync_copy(s_sh.at[sl], s_out.at[cid, sl])
    pltpu.sync_copy(deg_sh.at[sl], deg_out.at[cid, sl])


def _make_e1():
  return pl.kernel(
    _e1_body,
    out_type=[
        jax.ShapeDtypeStruct((EE,), jnp.float32),
        jax.ShapeDtypeStruct((NUM_SC, NN), jnp.float32),
        jax.ShapeDtypeStruct((NUM_SC, NN), jnp.float32),
    ],
    mesh=plsc.VectorSubcoreMesh(core_axis_name="c", subcore_axis_name="s"),
    scratch_types=[
        pltpu.VMEM((E1_BLK,), jnp.int32),
        pltpu.VMEM((E1_BLK,), jnp.int32),
        pltpu.VMEM((E1_BLK,), jnp.int32),
        pltpu.VMEM((E1_BLK // 128, 128), jnp.int32),
        pltpu.VMEM((E1_BLK // 128, 128), jnp.int32),
        pltpu.VMEM((E1_BLK // 128, 128), jnp.int32),
        pltpu.VMEM((E1_BLK,), jnp.float32),
        pltpu.VMEM((E1_BLK,), jnp.float32),
        pltpu.VMEM((E1_BLK,), jnp.float32),
        pltpu.VMEM((E1_BLK,), jnp.float32),
        pltpu.VMEM((NPT,), jnp.float32),
        pltpu.VMEM_SHARED((NN,), jnp.float32),
        pltpu.VMEM_SHARED((NN,), jnp.float32),
        pltpu.SemaphoreType.DMA,
    ],
  )


def _e2_body(
    hall_hbm, src_hbm, dst_hbm, et_hbm, e_hbm,
    out_hbm,
    src_v, dst_v, et_v, idxh_v, dstw_v, e_v, rows_v, zbuf,
    out_sh, sem,
):
    cid = lax.axis_index("c")
    sid = lax.axis_index("s")
    base0 = (cid * NUM_TILES + sid) * TILE_E

    def zf_body(i, _):
        zbuf[i // 8, pl.ds((i % 8) * 16, 16)] = jnp.zeros((16,), jnp.float32)
        return 0

    lax.fori_loop(0, 16 * DD // 16, zf_body, 0)
    nz = NPT // 16

    def zc_body(i, _):
        pltpu.sync_copy(zbuf, out_sh.at[pl.ds(sid * NPT + i * 16, 16), :])
        return 0

    lax.fori_loop(0, nz, zc_body, 0)
    plsc.subcore_barrier()

    def blk_body(blk, _):
        base = base0 + blk * E2_BLK
        pltpu.sync_copy(src_hbm.at[pl.ds(base, E2_BLK)], src_v)
        pltpu.sync_copy(dst_hbm.at[pl.ds(base, E2_BLK)], dst_v)
        pltpu.sync_copy(et_hbm.at[pl.ds(base, E2_BLK)], et_v)
        pltpu.sync_copy(e_hbm.at[pl.ds(base, E2_BLK)], e_v)

        def idx_body(i, _):
            sl = pl.ds(i * 16, 16)
            idxh_v[0, sl] = et_v[sl] * NN + src_v[sl]
            dstw_v[0, sl] = dst_v[sl]
            return 0

        lax.fori_loop(0, E2_BLK // 16, idx_body, 0)

        pltpu.async_copy(hall_hbm.at[idxh_v.at[0]], rows_v, sem).wait()

        def scale_body(g, _):
            cf16 = e_v[pl.ds(g * 16, 16)]
            for jj in range(16):
                b = g * 16 + jj
                cf = cf16[jj]
                for j in range(DD // 16):
                    sl2 = pl.ds(j * 16, 16)
                    rows_v[b, sl2] = rows_v[b, sl2] * cf
            return 0

        lax.fori_loop(0, E2_BLK // 16, scale_body, 0)

        pltpu.sync_copy(rows_v, out_sh.at[dstw_v.at[0]], add=True)
        return 0

    lax.fori_loop(0, TILE_E // E2_BLK, blk_body, 0)
    plsc.subcore_barrier()

    pltpu.sync_copy(
        out_sh.at[pl.ds(sid * NPT, NPT), :],
        out_hbm.at[cid, pl.ds(sid * NPT, NPT), :],
    )


def _make_e2():
  return pl.kernel(
    _e2_body,
    out_type=[
        jax.ShapeDtypeStruct((NUM_SC, NN, DD), jnp.float32),
    ],
    mesh=plsc.VectorSubcoreMesh(core_axis_name="c", subcore_axis_name="s"),
    scratch_types=[
        pltpu.VMEM((E2_BLK,), jnp.int32),
        pltpu.VMEM((E2_BLK,), jnp.int32),
        pltpu.VMEM((E2_BLK,), jnp.int32),
        pltpu.VMEM((1, E2_BLK), jnp.int32),
        pltpu.VMEM((1, E2_BLK), jnp.int32),
        pltpu.VMEM((E2_BLK,), jnp.float32),
        pltpu.VMEM((E2_BLK, DD), jnp.float32),
        pltpu.VMEM((16, DD), jnp.float32),
        pltpu.VMEM_SHARED((NN, DD), jnp.float32),
        pltpu.SemaphoreType.DMA,
    ],
  )


def kernel(
    x0, x1, x2, Pw0, Pb0, Pw1, Pb1, Pw2, Pb2,
    W1, q1, k1, b1, W2, q2, k2, b2,
    S1w, S1b, L1w, L1b, L2w, L2b, edge_index, edge_type,
):
    src = edge_index[0]
    dst = edge_index[1]
    et = edge_type

    l2w_pad = jnp.pad(L2w, ((0, 0), (0, DD - L2w.shape[1])))
    l2b_pad = jnp.pad(L2b, (0, DD - L2b.shape[0]))

    xh0, sl0 = _proj(x0, Pw0, Pb0, S1w[0], S1b[0])
    xh1, sl1 = _proj(x1, Pw1, Pb1, S1w[1], S1b[1])
    xh2, sl2 = _proj(x2, Pw2, Pb2, S1w[2], S1b[2])
    xh = jnp.concatenate([xh0, xh1, xh2], axis=0)
    sl = jnp.concatenate([sl0, sl1, sl2], axis=0)

    e1_call = _make_e1()
    e2_call = _make_e2()

    hall1, sq1, sk1 = _hall_scores(xh, W1, q1, k1)
    e1, s1p, degp = e1_call(
        sq1.reshape(RR * NN), sk1.reshape(RR * NN), src, dst, et
    )
    (op1,) = e2_call(hall1.reshape(RR * NN, DD), src, dst, et, e1)
    h1 = _combine(op1, s1p, degp, b1, sl)

    hall2, sq2, sk2 = _hall_scores(h1, W2, q2, k2)
    e2, s2p, _deg2 = e1_call(
        sq2.reshape(RR * NN), sk2.reshape(RR * NN), src, dst, et
    )
    (op2,) = e2_call(hall2.reshape(RR * NN, DD), src, dst, et, e2)

    y = _head(op2, s2p, degp, b2, sl, L1w, L1b, l2w_pad, l2b_pad)
    return y[:, : L2w.shape[1]]


# speedup vs baseline: 44.4617x; 1.3270x over previous
import jax
import jax.numpy as jnp
from jax import lax
from jax.experimental import pallas as pl
from jax.experimental.pallas import tpu as pltpu
from jax.experimental.pallas import tpu_sc as plsc

N0 = 4096
NUM_OMICS = 3
NN = N0 * NUM_OMICS
EE = 393216
RR = 6
DD = 128
SQK_STRIDE = 16

NUM_SC = 2
NUM_TILES = 16
NW = NUM_SC * NUM_TILES
TILE_E = EE // NW

E1_BLK = 1024
E2_BLK = 128
NPT = NN // NUM_TILES


def _elu(x):
    return jnp.where(x > 0, x, jnp.exp(jnp.minimum(x, 0.0)) - 1.0)


def _proj_body(x_ref, pw_ref, pb_ref, s1w_ref, s1b_ref, xh_ref, sl_ref):
    p = jnp.dot(x_ref[...], pw_ref[...], preferred_element_type=jnp.float32)
    p = p + pb_ref[...]
    xh = _elu(p)
    xh_ref[...] = xh
    sl_ref[...] = (
        jnp.dot(xh, s1w_ref[...], preferred_element_type=jnp.float32) + s1b_ref[...]
    )


def _proj(x, pw, pb, s1w, s1b):
    n, k = x.shape
    bn = 256
    return pl.pallas_call(
        _proj_body,
        grid=(n // bn,),
        in_specs=[
            pl.BlockSpec((bn, k), lambda b: (b, 0)),
            pl.BlockSpec((k, DD), lambda b: (0, 0)),
            pl.BlockSpec((1, DD), lambda b: (0, 0)),
            pl.BlockSpec((DD, DD), lambda b: (0, 0)),
            pl.BlockSpec((1, DD), lambda b: (0, 0)),
        ],
        out_specs=[
            pl.BlockSpec((bn, DD), lambda b: (b, 0)),
            pl.BlockSpec((bn, DD), lambda b: (b, 0)),
        ],
        out_shape=[
            jax.ShapeDtypeStruct((n, DD), jnp.float32),
            jax.ShapeDtypeStruct((n, DD), jnp.float32),
        ],
    )(x, pw, pb.reshape(1, DD), s1w, s1b.reshape(1, DD))


def _hall_body(x_ref, w_ref, q_ref, k_ref, h_ref, sq_ref, sk_ref):
    h = jnp.dot(x_ref[...], w_ref[0], preferred_element_type=jnp.float32)
    h_ref[0] = h
    sq_ref[0] = jnp.sum(h * q_ref[0], axis=1, keepdims=True)
    sk_ref[0] = jnp.sum(h * k_ref[0], axis=1, keepdims=True)


def _hall_scores(x, w, q, k):
    bn = 256
    return pl.pallas_call(
        _hall_body,
        grid=(RR, NN // bn),
        in_specs=[
            pl.BlockSpec((bn, DD), lambda r, b: (b, 0)),
            pl.BlockSpec((1, DD, DD), lambda r, b: (r, 0, 0)),
            pl.BlockSpec((1, 1, DD), lambda r, b: (r, 0, 0)),
            pl.BlockSpec((1, 1, DD), lambda r, b: (r, 0, 0)),
        ],
        out_specs=[
            pl.BlockSpec((1, bn, DD), lambda r, b: (r, b, 0)),
            pl.BlockSpec((1, bn, 1), lambda r, b: (r, b, 0)),
            pl.BlockSpec((1, bn, 1), lambda r, b: (r, b, 0)),
        ],
        out_shape=[
            jax.ShapeDtypeStruct((RR, NN, DD), jnp.float32),
            jax.ShapeDtypeStruct((RR, NN, 1), jnp.float32),
            jax.ShapeDtypeStruct((RR, NN, 1), jnp.float32),
        ],
    )(x, w, q.reshape(RR, 1, DD), k.reshape(RR, 1, DD))


def _combine_body(op_ref, sp_ref, dp_ref, b_ref, sl_ref, out_ref):
    s = sp_ref[0] + sp_ref[1]
    deg = dp_ref[0] + dp_ref[1]
    w = deg / (s + 1e-16)
    h = w * (op_ref[0] + op_ref[1]) + b_ref[...] + sl_ref[...]
    out_ref[...] = _elu(h)


def _combine(outp, s_part, deg_part, b, sl):
    bn = 256
    return pl.pallas_call(
        _combine_body,
        grid=(NN // bn,),
        in_specs=[
            pl.BlockSpec((2, bn, DD), lambda b_: (0, b_, 0)),
            pl.BlockSpec((2, bn, 1), lambda b_: (0, b_, 0)),
            pl.BlockSpec((2, bn, 1), lambda b_: (0, b_, 0)),
            pl.BlockSpec((1, DD), lambda b_: (0, 0)),
            pl.BlockSpec((bn, DD), lambda b_: (b_, 0)),
        ],
        out_specs=pl.BlockSpec((bn, DD), lambda b_: (b_, 0)),
        out_shape=jax.ShapeDtypeStruct((NN, DD), jnp.float32),
    )(
        outp,
        s_part.reshape(2, NN, 1),
        deg_part.reshape(2, NN, 1),
        b.reshape(1, DD),
        sl,
    )


def _head_body(
    op_ref, sp_ref, dp_ref, b_ref, sl_ref, l1w_ref, l1b_ref, l2w_ref, l2b_ref, y_ref
):
    s = sp_ref[0] + sp_ref[1]
    deg = dp_ref[0] + dp_ref[1]
    w = deg / (s + 1e-16)
    h = _elu(w * (op_ref[0] + op_ref[1]) + b_ref[...] + sl_ref[...])
    z = _elu(jnp.dot(h, l1w_ref[...], preferred_element_type=jnp.float32) + l1b_ref[...])
    y_ref[...] = (
        jnp.dot(z, l2w_ref[...], preferred_element_type=jnp.float32) + l2b_ref[...]
    )


def _head(outp, s_part, deg_part, b, sl, l1w, l1b, l2w_pad, l2b_pad):
    bn = 256
    return pl.pallas_call(
        _head_body,
        grid=(N0 // bn,),
        in_specs=[
            pl.BlockSpec((2, bn, DD), lambda b_: (0, b_, 0)),
            pl.BlockSpec((2, bn, 1), lambda b_: (0, b_, 0)),
            pl.BlockSpec((2, bn, 1), lambda b_: (0, b_, 0)),
            pl.BlockSpec((1, DD), lambda b_: (0, 0)),
            pl.BlockSpec((bn, DD), lambda b_: (b_, 0)),
            pl.BlockSpec((DD, DD), lambda b_: (0, 0)),
            pl.BlockSpec((1, DD), lambda b_: (0, 0)),
            pl.BlockSpec((DD, DD), lambda b_: (0, 0)),
            pl.BlockSpec((1, DD), lambda b_: (0, 0)),
        ],
        out_specs=pl.BlockSpec((bn, DD), lambda b_: (b_, 0)),
        out_shape=jax.ShapeDtypeStruct((N0, DD), jnp.float32),
    )(
        outp,
        s_part.reshape(2, NN, 1),
        deg_part.reshape(2, NN, 1),
        b.reshape(1, DD),
        sl,
        l1w,
        l1b.reshape(1, DD),
        l2w_pad,
        l2b_pad.reshape(1, DD),
    )


E2_BLK = 64
NIR = E1_BLK // E2_BLK


def _zero_fill(buf, nwords):
    z = jnp.zeros((16,), jnp.float32)

    def body(i, _):
        buf[pl.ds(i * 16, 16)] = z
        return 0

    lax.fori_loop(0, nwords // 16, body, 0)


def _edge_body(
    sq_hbm, sk_hbm, hall_hbm, src_hbm, dst_hbm, et_hbm,
    s_out, deg_out, out_hbm,
    src_v, dst_v, et_v, idxd_v, idxs_v, dstw_v,
    val_d, val_s, e_v, ones_v, rows0, rows1,
    s_sh, deg_sh, out_sh,
    sem_q, sem_sd, sem_g0, sem_g1, sem_s0, sem_s1, sem_z,
):
    cid = lax.axis_index("c")
    sid = lax.axis_index("s")
    base0 = (cid * NUM_TILES + sid) * TILE_E

    def zf(i, _):
        rows0[i // 8, pl.ds((i % 8) * 16, 16)] = jnp.zeros((16,), jnp.float32)
        return 0

    lax.fori_loop(0, E2_BLK * 8, zf, 0)
    _zero_fill(e_v, E1_BLK)

    def fo(i, _):
        ones_v[pl.ds(i * 16, 16)] = jnp.ones((16,), jnp.float32)
        return 0

    lax.fori_loop(0, E1_BLK // 16, fo, 0)

    zc = [
        pltpu.async_copy(
            rows0, out_sh.at[pl.ds(sid * NPT + i * E2_BLK, E2_BLK), :], sem_z
        )
        for i in range(NPT // E2_BLK)
    ]
    for cp in zc:
        cp.wait()
    pltpu.sync_copy(e_v.at[pl.ds(0, NPT)], s_sh.at[pl.ds(sid * NPT, NPT)])
    pltpu.sync_copy(e_v.at[pl.ds(0, NPT)], deg_sh.at[pl.ds(sid * NPT, NPT)])
    plsc.subcore_barrier()

    def _scale(rows, eoff):
        def sc(g, _):
            cf16 = e_v[pl.ds(eoff + g * 16, 16)]
            for jj in range(16):
                cf = cf16[jj]
                for j in range(DD // 16):
                    sl2 = pl.ds(j * 16, 16)
                    rows[g * 16 + jj, sl2] = rows[g * 16 + jj, sl2] * cf
            return 0

        lax.fori_loop(0, E2_BLK // 16, sc, 0)

    def sb_body(sb, _):
        base = base0 + sb * E1_BLK
        pltpu.sync_copy(src_hbm.at[pl.ds(base, E1_BLK)], src_v)
        pltpu.sync_copy(dst_hbm.at[pl.ds(base, E1_BLK)], dst_v)
        pltpu.sync_copy(et_hbm.at[pl.ds(base, E1_BLK)], et_v)

        def ib(i, _):
            sl = pl.ds(i * 16, 16)
            d16 = dst_v[sl]
            enn = et_v[sl] * NN
            r = i // 4
            o = (i % 4) * 16
            idxd_v[r, pl.ds(o, 16)] = enn + d16
            idxs_v[r, pl.ds(o, 16)] = enn + src_v[sl]
            dstw_v[r, pl.ds(o, 16)] = d16
            return 0

        lax.fori_loop(0, E1_BLK // 16, ib, 0)

        qc = []
        for c in range(NIR):
            qc.append(pltpu.async_copy(
                sq_hbm.at[idxd_v.at[c]], val_d.at[pl.ds(c * E2_BLK, E2_BLK)], sem_q))
            qc.append(pltpu.async_copy(
                sk_hbm.at[idxs_v.at[c]], val_s.at[pl.ds(c * E2_BLK, E2_BLK)], sem_q))
        for cp in qc:
            cp.wait()

        def lg(i, _):
            sl = pl.ds(i * 16, 16)
            l = val_d[sl] + val_s[sl]
            l = jnp.maximum(l, 0.2 * l)
            e_v[sl] = jnp.exp(l)
            return 0

        lax.fori_loop(0, E1_BLK // 16, lg, 0)

        sd = []
        for c in range(NIR):
            sl = pl.ds(c * E2_BLK, E2_BLK)
            sd.append(pltpu.async_copy(
                e_v.at[sl], s_sh.at[dstw_v.at[c]], sem_sd, add=True))
            sd.append(pltpu.async_copy(
                ones_v.at[sl], deg_sh.at[dstw_v.at[c]], sem_sd, add=True))

        def pair_body(jp, _):
            r0 = 2 * jp
            r1 = r0 + 1
            g0 = pltpu.async_copy(hall_hbm.at[idxs_v.at[r0]], rows0, sem_g0)
            g1 = pltpu.async_copy(hall_hbm.at[idxs_v.at[r1]], rows1, sem_g1)
            g0.wait()
            _scale(rows0, r0 * E2_BLK)
            s0 = pltpu.async_copy(rows0, out_sh.at[dstw_v.at[r0]], sem_s0, add=True)
            g1.wait()
            _scale(rows1, r1 * E2_BLK)
            s1 = pltpu.async_copy(rows1, out_sh.at[dstw_v.at[r1]], sem_s1, add=True)
            s0.wait()
            s1.wait()
            return 0

        lax.fori_loop(0, NIR // 2, pair_body, 0)

        for cp in sd:
            cp.wait()
        return 0

    lax.fori_loop(0, TILE_E // E1_BLK, sb_body, 0)
    plsc.subcore_barrier()

    sl = pl.ds(sid * NPT, NPT)
    pltpu.sync_copy(s_sh.at[sl], s_out.at[cid, sl])
    pltpu.sync_copy(deg_sh.at[sl], deg_out.at[cid, sl])
    pltpu.sync_copy(out_sh.at[sl, :], out_hbm.at[cid, sl, :])


def _make_edge():
  return pl.kernel(
    _edge_body,
    out_type=[
        jax.ShapeDtypeStruct((NUM_SC, NN), jnp.float32),
        jax.ShapeDtypeStruct((NUM_SC, NN), jnp.float32),
        jax.ShapeDtypeStruct((NUM_SC, NN, DD), jnp.float32),
    ],
    mesh=plsc.VectorSubcoreMesh(core_axis_name="c", subcore_axis_name="s"),
    scratch_types=[
        pltpu.VMEM((E1_BLK,), jnp.int32),
        pltpu.VMEM((E1_BLK,), jnp.int32),
        pltpu.VMEM((E1_BLK,), jnp.int32),
        pltpu.VMEM((NIR, E2_BLK), jnp.int32),
        pltpu.VMEM((NIR, E2_BLK), jnp.int32),
        pltpu.VMEM((NIR, E2_BLK), jnp.int32),
        pltpu.VMEM((E1_BLK,), jnp.float32),
        pltpu.VMEM((E1_BLK,), jnp.float32),
        pltpu.VMEM((E1_BLK,), jnp.float32),
        pltpu.VMEM((E1_BLK,), jnp.float32),
        pltpu.VMEM((E2_BLK, DD), jnp.float32),
        pltpu.VMEM((E2_BLK, DD), jnp.float32),
        pltpu.VMEM_SHARED((NN,), jnp.float32),
        pltpu.VMEM_SHARED((NN,), jnp.float32),
        pltpu.VMEM_SHARED((NN, DD), jnp.float32),
        pltpu.SemaphoreType.DMA,
        pltpu.SemaphoreType.DMA,
        pltpu.SemaphoreType.DMA,
        pltpu.SemaphoreType.DMA,
        pltpu.SemaphoreType.DMA,
        pltpu.SemaphoreType.DMA,
        pltpu.SemaphoreType.DMA,
    ],
  )


def kernel(
    x0, x1, x2, Pw0, Pb0, Pw1, Pb1, Pw2, Pb2,
    W1, q1, k1, b1, W2, q2, k2, b2,
    S1w, S1b, L1w, L1b, L2w, L2b, edge_index, edge_type,
):
    src = edge_index[0]
    dst = edge_index[1]
    et = edge_type

    l2w_pad = jnp.pad(L2w, ((0, 0), (0, DD - L2w.shape[1])))
    l2b_pad = jnp.pad(L2b, (0, DD - L2b.shape[0]))

    xh0, sl0 = _proj(x0, Pw0, Pb0, S1w[0], S1b[0])
    xh1, sl1 = _proj(x1, Pw1, Pb1, S1w[1], S1b[1])
    xh2, sl2 = _proj(x2, Pw2, Pb2, S1w[2], S1b[2])
    xh = jnp.concatenate([xh0, xh1, xh2], axis=0)
    sl = jnp.concatenate([sl0, sl1, sl2], axis=0)

    edge_call = _make_edge()

    hall1, sq1, sk1 = _hall_scores(xh, W1, q1, k1)
    s1p, degp, op1 = edge_call(
        sq1.reshape(RR * NN), sk1.reshape(RR * NN),
        hall1.reshape(RR * NN, DD), src, dst, et,
    )
    h1 = _combine(op1, s1p, degp, b1, sl)

    hall2, sq2, sk2 = _hall_scores(h1, W2, q2, k2)
    s2p, _deg2, op2 = edge_call(
        sq2.reshape(RR * NN), sk2.reshape(RR * NN),
        hall2.reshape(RR * NN, DD), src, dst, et,
    )

    y = _head(op2, s2p, degp, b2, sl, L1w, L1b, l2w_pad, l2b_pad)
    return y[:, : L2w.shape[1]]
